# baseline (speedup 1.0000x reference)
import jax
import jax.numpy as jnp
from jax.experimental import pallas as pl
from jax.experimental.pallas import tpu as pltpu


def _add_body(ids_ref, hid_ref, emb_ref, gate_ref, out_ref):
    g = jnp.tanh(gate_ref[...])
    out_ref[...] = hid_ref[...] + emb_ref[...] * g


def kernel(hidden_state, aspect_ratio_ids, embedding_table, gate):
    B, T, P, H = hidden_state.shape
    emb = embedding_table.reshape(-1, T, 1, H)
    ids = aspect_ratio_ids.astype(jnp.int32)
    gate2d = gate.reshape(1, 1)

    grid_spec = pltpu.PrefetchScalarGridSpec(
        num_scalar_prefetch=1,
        grid=(B, T),
        in_specs=[
            pl.BlockSpec((1, 1, P, H), lambda b, t, ids_ref: (b, t, 0, 0)),
            pl.BlockSpec((1, 1, 1, H), lambda b, t, ids_ref: (ids_ref[b], t, 0, 0)),
            pl.BlockSpec((1, 1), lambda b, t, ids_ref: (0, 0)),
        ],
        out_specs=pl.BlockSpec((1, 1, P, H), lambda b, t, ids_ref: (b, t, 0, 0)),
    )
    return pl.pallas_call(
        _add_body,
        grid_spec=grid_spec,
        out_shape=jax.ShapeDtypeStruct((B, T, P, H), hidden_state.dtype),
    )(ids, hidden_state, emb, gate2d)
